# baseline (speedup 1.0000x reference)
import functools

import jax
import jax.numpy as jnp
from jax import lax
from jax.experimental import pallas as pl
from jax.experimental.pallas import tpu as pltpu
from jax.experimental.pallas import tpu_sc as plsc

NCORES = 2
NTILES = 16
NW = NCORES * NTILES
CS = 128
NBUF = 2
AGG_SPLIT0 = 0.48


def _sc_degree(col3, ones_h, zeros_h, n_pad, nch, rpt, d):
    mesh = plsc.VectorSubcoreMesh(core_axis_name="c", subcore_axis_name="s")

    @functools.partial(
        pl.kernel,
        out_type=jax.ShapeDtypeStruct((NCORES, n_pad, d), jnp.float32),
        mesh=mesh,
        scratch_types=[
            pltpu.VMEM((nch, CS), jnp.int32),
            pltpu.VMEM((CS, d), jnp.float32),
            pltpu.VMEM_SHARED((n_pad, d), jnp.float32),
        ] + [pltpu.SemaphoreType.DMA] * NBUF,
    )
    def deg_kernel(col_h, ones_hbm, zeros_hbm, out_h, idx_v, ones_v, acc,
                   *sems):
        cid = lax.axis_index("c")
        sid = lax.axis_index("s")
        wid = cid * NTILES + sid
        pltpu.sync_copy(col_h.at[wid], idx_v)
        pltpu.sync_copy(ones_hbm, ones_v)
        pltpu.sync_copy(zeros_hbm, acc.at[pl.ds(sid * rpt, rpt)])
        plsc.subcore_barrier()

        def body(p, carry):
            base = p * NBUF
            for i in range(NBUF):
                @pl.when(p > 0)
                def _wait():
                    pltpu.make_async_copy(ones_hbm, ones_v, sems[i]).wait()
                pltpu.async_copy(ones_v, acc.at[idx_v.at[base + i]], sems[i],
                                 add=True)
            return carry

        lax.fori_loop(0, nch // NBUF, body, 0)
        for i in range(NBUF):
            pltpu.make_async_copy(ones_hbm, ones_v, sems[i]).wait()
        plsc.subcore_barrier()
        pltpu.sync_copy(acc.at[pl.ds(sid * rpt, rpt)],
                        out_h.at[cid, pl.ds(sid * rpt, rpt)])

    return deg_kernel(col3, ones_h, zeros_h)


def _sc_aggregate(xs, row3, col3, zeros_h, n_pad, nch, rpt, d):
    mesh = plsc.VectorSubcoreMesh(core_axis_name="c", subcore_axis_name="s")

    @functools.partial(
        pl.kernel,
        out_type=jax.ShapeDtypeStruct((NCORES, n_pad, d), jnp.float32),
        mesh=mesh,
        scratch_types=[
            pltpu.VMEM((nch, CS), jnp.int32),
            pltpu.VMEM((nch, CS), jnp.int32),
            pltpu.VMEM((CS, d), jnp.float32),
            pltpu.SemaphoreType.DMA,
            pltpu.VMEM_SHARED((n_pad, d), jnp.float32),
        ],
    )
    def agg_kernel(xs_h, row_h, col_h, zeros_hbm, out_h,
                   ridx, cidx, buf, sem, acc):
        cid = lax.axis_index("c")
        sid = lax.axis_index("s")
        wid = cid * NTILES + sid
        pltpu.sync_copy(row_h.at[wid], ridx)
        pltpu.sync_copy(col_h.at[wid], cidx)
        pltpu.sync_copy(zeros_hbm, acc.at[pl.ds(sid * rpt, rpt)])
        plsc.subcore_barrier()

        def body(j, carry):
            pltpu.async_copy(xs_h.at[ridx.at[j]], buf, sem).wait()
            pltpu.sync_copy(buf, acc.at[cidx.at[j]], add=True)
            return carry

        lax.fori_loop(0, nch, body, 0)
        plsc.subcore_barrier()
        pltpu.sync_copy(acc.at[pl.ds(sid * rpt, rpt)],
                        out_h.at[cid, pl.ds(sid * rpt, rpt)])

    return agg_kernel(xs, row3, col3, zeros_h)


def _scale_body(x_ref, dg_ref, o_ref):
    cnt = dg_ref[0, :, 0:1] + dg_ref[1, :, 0:1] + 1.0
    o_ref[...] = x_ref[...] * lax.rsqrt(cnt)


def _tc_scale(x, degs, bl):
    n, d = x.shape
    return pl.pallas_call(
        _scale_body,
        grid=(n // bl,),
        in_specs=[
            pl.BlockSpec((bl, d), lambda i: (i, 0)),
            pl.BlockSpec((NCORES, bl, 16), lambda i: (0, i, 0)),
        ],
        out_specs=pl.BlockSpec((bl, d), lambda i: (i, 0)),
        out_shape=jax.ShapeDtypeStruct((n, d), jnp.float32),
    )(x, degs)


def _dense_body(nk, x_ref, xs_ref, p_ref, dg_ref, ws_ref, bs_ref, wd_ref,
                bd_ref, o_ref):
    x = x_ref[...]
    cnt = dg_ref[0, :, 0:1] + dg_ref[1, :, 0:1] + 1.0
    dinv = lax.rsqrt(cnt)
    agg = (p_ref[0] + p_ref[1] + xs_ref[...]) * dinv
    logits = jnp.dot(x, wd_ref[...], preferred_element_type=jnp.float32)
    logits = logits + bd_ref[...]
    lane = lax.broadcasted_iota(jnp.int32, logits.shape, 1)
    valid = lane < nk
    logits = jnp.where(valid, logits, -1e30)
    m = jnp.max(logits, axis=-1, keepdims=True)
    e = jnp.where(valid, jnp.exp(logits - m), 0.0)
    coeff = e / jnp.sum(e, axis=-1, keepdims=True)
    acc = jnp.zeros_like(x)
    for k in range(nk):
        f = jnp.dot(agg, ws_ref[k], preferred_element_type=jnp.float32)
        f = jnp.maximum(f + bs_ref[k], 0.0)
        acc = acc + f * coeff[:, k:k + 1]
    o_ref[...] = acc


def _tc_dense(x, xs, P, degs, Ws, bs, Wdp, bdp, bl):
    n, d = x.shape
    nk = Ws.shape[0]
    return pl.pallas_call(
        functools.partial(_dense_body, nk),
        grid=(n // bl,),
        in_specs=[
            pl.BlockSpec((bl, d), lambda i: (i, 0)),
            pl.BlockSpec((bl, d), lambda i: (i, 0)),
            pl.BlockSpec((NCORES, bl, d), lambda i: (0, i, 0)),
            pl.BlockSpec((NCORES, bl, 16), lambda i: (0, i, 0)),
            pl.BlockSpec((nk, d, d), lambda i: (0, 0, 0)),
            pl.BlockSpec((nk, d), lambda i: (0, 0)),
            pl.BlockSpec((d, d), lambda i: (0, 0)),
            pl.BlockSpec((1, d), lambda i: (0, 0)),
        ],
        out_specs=pl.BlockSpec((bl, d), lambda i: (i, 0)),
        out_shape=jax.ShapeDtypeStruct((n, d), jnp.float32),
    )(x, xs, P, degs, Ws, bs, Wdp, bdp)


def kernel(x, edge_index, Ws, bs, Wd, bd):
    n, d = x.shape
    e = edge_index.shape[1]
    nk = Ws.shape[0]
    na = Wd.shape[1]

    ept = -(-e // (NW * CS * 4)) * CS * 4
    e_pad = ept * NW
    nch = ept // CS
    n_pad = -(-(n + CS) // (NTILES * 8)) * (NTILES * 8)
    rpt = n_pad // NTILES

    row = edge_index[0].astype(jnp.int32)
    col = edge_index[1].astype(jnp.int32)
    pad = e_pad - e
    row3 = jnp.concatenate([row, jnp.zeros((pad,), jnp.int32)]).reshape(NW, nch, CS)
    dump = n + (jnp.arange(pad, dtype=jnp.int32) % CS)
    col3 = jnp.concatenate([col, dump]).reshape(NW, nch, CS)
    ones_h = jnp.ones((CS, d), jnp.float32)
    zagg = jnp.zeros((rpt, d), jnp.float32)

    R = -(-e // (NTILES * CS))
    r0 = min(max(int(round(R * AGG_SPLIT0)), 1), R - 1)
    r1 = R - r0
    nmx = max(r0, r1)
    e_padA = R * NTILES * CS
    padA = e_padA - e
    dumpA = n + (jnp.arange(padA, dtype=jnp.int32) % CS)
    chr_ = jnp.concatenate([row, jnp.zeros((padA,), jnp.int32)]).reshape(-1, CS)
    chc_ = jnp.concatenate([col, dumpA]).reshape(-1, CS)
    dump_row = jnp.zeros((CS,), jnp.int32)
    dump_col = n + jnp.arange(CS, dtype=jnp.int32)

    def _core_layout(chunks, r_real, fill):
        part = chunks.reshape(NTILES, r_real, CS)
        if r_real == nmx:
            return part
        extra = jnp.broadcast_to(fill, (NTILES, nmx - r_real, CS))
        return jnp.concatenate([part, extra], axis=1)

    row3a = jnp.concatenate([
        _core_layout(chr_[:NTILES * r0], r0, dump_row),
        _core_layout(chr_[NTILES * r0:], r1, dump_row)], axis=0)
    col3a = jnp.concatenate([
        _core_layout(chc_[:NTILES * r0], r0, dump_col),
        _core_layout(chc_[NTILES * r0:], r1, dump_col)], axis=0)

    degc = _sc_degree(col3, ones_h, zagg, n_pad, nch, rpt, d)
    degs = degc[:, :n, :16]
    bl = 2000
    xs = _tc_scale(x, degs, bl)
    P = _sc_aggregate(xs, row3a, col3a, zagg, n_pad, nmx, rpt, d)[:, :n, :]

    Wdp = jnp.zeros((d, d), jnp.float32).at[:, :na].set(Wd)
    bdp = jnp.zeros((1, d), jnp.float32).at[0, :na].set(bd)
    return _tc_dense(x, xs, P, degs, Ws, bs, Wdp, bdp, bl)


# speedup vs baseline: 2.8247x; 2.8247x over previous
import functools

import jax
import jax.numpy as jnp
from jax import lax
from jax.experimental import pallas as pl
from jax.experimental.pallas import tpu as pltpu
from jax.experimental.pallas import tpu_sc as plsc

NCORES = 2
NTILES = 16
NW = NCORES * NTILES
CS = 128
NBUF = 2
AGG_SPLIT0 = 0.5


def _sc_degree(col3, ones_h, zeros_h, n_pad, nch, rpt, d):
    mesh = plsc.VectorSubcoreMesh(core_axis_name="c", subcore_axis_name="s")

    @functools.partial(
        pl.kernel,
        out_type=jax.ShapeDtypeStruct((NCORES, n_pad, d), jnp.float32),
        mesh=mesh,
        scratch_types=[
            pltpu.VMEM((nch, CS), jnp.int32),
            pltpu.VMEM((CS, d), jnp.float32),
            pltpu.VMEM_SHARED((n_pad, d), jnp.float32),
        ] + [pltpu.SemaphoreType.DMA] * NBUF,
    )
    def deg_kernel(col_h, ones_hbm, zeros_hbm, out_h, idx_v, ones_v, acc,
                   *sems):
        cid = lax.axis_index("c")
        sid = lax.axis_index("s")
        wid = cid * NTILES + sid
        pltpu.sync_copy(col_h.at[wid], idx_v)
        pltpu.sync_copy(ones_hbm, ones_v)
        pltpu.sync_copy(zeros_hbm, acc.at[pl.ds(sid * rpt, rpt)])
        plsc.subcore_barrier()

        def body(p, carry):
            base = p * NBUF
            for i in range(NBUF):
                @pl.when(p > 0)
                def _wait():
                    pltpu.make_async_copy(ones_hbm, ones_v, sems[i]).wait()
                pltpu.async_copy(ones_v, acc.at[idx_v.at[base + i]], sems[i],
                                 add=True)
            return carry

        lax.fori_loop(0, nch // NBUF, body, 0)
        for i in range(NBUF):
            pltpu.make_async_copy(ones_hbm, ones_v, sems[i]).wait()
        plsc.subcore_barrier()
        pltpu.sync_copy(acc.at[pl.ds(sid * rpt, rpt)],
                        out_h.at[cid, pl.ds(sid * rpt, rpt)])

    return deg_kernel(col3, ones_h, zeros_h)


def _sc_aggregate(xs, row3, col3, zeros_h, n_pad, nch, rpt, d):
    mesh = plsc.VectorSubcoreMesh(core_axis_name="c", subcore_axis_name="s")

    @functools.partial(
        pl.kernel,
        out_type=jax.ShapeDtypeStruct((NCORES, n_pad, d), jnp.float32),
        mesh=mesh,
        scratch_types=[
            pltpu.VMEM((nch, CS), jnp.int32),
            pltpu.VMEM((nch, CS), jnp.int32),
            pltpu.VMEM((CS, d), jnp.float32),
            pltpu.SemaphoreType.DMA,
            pltpu.VMEM_SHARED((n_pad, d), jnp.float32),
        ],
    )
    def agg_kernel(xs_h, row_h, col_h, zeros_hbm, out_h,
                   ridx, cidx, buf, sem, acc):
        cid = lax.axis_index("c")
        sid = lax.axis_index("s")
        wid = cid * NTILES + sid
        pltpu.sync_copy(row_h.at[wid], ridx)
        pltpu.sync_copy(col_h.at[wid], cidx)
        pltpu.sync_copy(zeros_hbm, acc.at[pl.ds(sid * rpt, rpt)])
        plsc.subcore_barrier()

        def body(j, carry):
            pltpu.async_copy(xs_h.at[ridx.at[j]], buf, sem).wait()
            pltpu.sync_copy(buf, acc.at[cidx.at[j]], add=True)
            return carry

        lax.fori_loop(0, nch, body, 0)
        plsc.subcore_barrier()
        pltpu.sync_copy(acc.at[pl.ds(sid * rpt, rpt)],
                        out_h.at[cid, pl.ds(sid * rpt, rpt)])

    return agg_kernel(xs, row3, col3, zeros_h)


def _scale_body(x_ref, dg_ref, o_ref):
    cnt = dg_ref[0, :, 0:1] + dg_ref[1, :, 0:1] + 1.0
    o_ref[...] = x_ref[...] * lax.rsqrt(cnt)


def _tc_scale(x, degs, bl):
    n, d = x.shape
    return pl.pallas_call(
        _scale_body,
        grid=(n // bl,),
        in_specs=[
            pl.BlockSpec((bl, d), lambda i: (i, 0)),
            pl.BlockSpec((NCORES, bl, 16), lambda i: (0, i, 0)),
        ],
        out_specs=pl.BlockSpec((bl, d), lambda i: (i, 0)),
        out_shape=jax.ShapeDtypeStruct((n, d), jnp.float32),
    )(x, degs)


def _dense_body(nk, x_ref, xs_ref, p_ref, dg_ref, ws_ref, bs_ref, wd_ref,
                bd_ref, o_ref):
    x = x_ref[...]
    cnt = dg_ref[0, :, 0:1] + dg_ref[1, :, 0:1] + 1.0
    dinv = lax.rsqrt(cnt)
    agg = (p_ref[0] + p_ref[1] + xs_ref[...]) * dinv
    logits = jnp.dot(x, wd_ref[...], preferred_element_type=jnp.float32)
    logits = logits + bd_ref[...]
    lane = lax.broadcasted_iota(jnp.int32, logits.shape, 1)
    valid = lane < nk
    logits = jnp.where(valid, logits, -1e30)
    m = jnp.max(logits, axis=-1, keepdims=True)
    e = jnp.where(valid, jnp.exp(logits - m), 0.0)
    coeff = e / jnp.sum(e, axis=-1, keepdims=True)
    acc = jnp.zeros_like(x)
    for k in range(nk):
        f = jnp.dot(agg, ws_ref[k], preferred_element_type=jnp.float32)
        f = jnp.maximum(f + bs_ref[k], 0.0)
        acc = acc + f * coeff[:, k:k + 1]
    o_ref[...] = acc


def _tc_dense(x, xs, P, degs, Ws, bs, Wdp, bdp, bl):
    n, d = x.shape
    nk = Ws.shape[0]
    return pl.pallas_call(
        functools.partial(_dense_body, nk),
        grid=(n // bl,),
        in_specs=[
            pl.BlockSpec((bl, d), lambda i: (i, 0)),
            pl.BlockSpec((bl, d), lambda i: (i, 0)),
            pl.BlockSpec((NCORES, bl, d), lambda i: (0, i, 0)),
            pl.BlockSpec((NCORES, bl, 16), lambda i: (0, i, 0)),
            pl.BlockSpec((nk, d, d), lambda i: (0, 0, 0)),
            pl.BlockSpec((nk, d), lambda i: (0, 0)),
            pl.BlockSpec((d, d), lambda i: (0, 0)),
            pl.BlockSpec((1, d), lambda i: (0, 0)),
        ],
        out_specs=pl.BlockSpec((bl, d), lambda i: (i, 0)),
        out_shape=jax.ShapeDtypeStruct((n, d), jnp.float32),
    )(x, xs, P, degs, Ws, bs, Wdp, bdp)


def kernel(x, edge_index, Ws, bs, Wd, bd):
    n, d = x.shape
    e = edge_index.shape[1]
    nk = Ws.shape[0]
    na = Wd.shape[1]

    ept = -(-e // (NW * CS * 4)) * CS * 4
    e_pad = ept * NW
    nch = ept // CS
    n_pad = -(-(n + CS) // (NTILES * 8)) * (NTILES * 8)
    rpt = n_pad // NTILES

    row = edge_index[0].astype(jnp.int32)
    col = edge_index[1].astype(jnp.int32)
    pad = e_pad - e
    row3 = jnp.concatenate([row, jnp.zeros((pad,), jnp.int32)]).reshape(NW, nch, CS)
    dump = n + (jnp.arange(pad, dtype=jnp.int32) % CS)
    col3 = jnp.concatenate([col, dump]).reshape(NW, nch, CS)
    ones_h = jnp.ones((CS, d), jnp.float32)
    zagg = jnp.zeros((rpt, d), jnp.float32)

    R = -(-e // (NTILES * CS))
    r0 = min(max(int(round(R * AGG_SPLIT0)), 1), R - 1)
    r1 = R - r0
    nmx = max(r0, r1)
    e_padA = R * NTILES * CS
    padA = e_padA - e
    dumpA = n + (jnp.arange(padA, dtype=jnp.int32) % CS)
    spreadA = jnp.arange(padA, dtype=jnp.int32) % CS
    chr_ = jnp.concatenate([row, spreadA]).reshape(-1, CS)
    chc_ = jnp.concatenate([col, dumpA]).reshape(-1, CS)
    dump_row = jnp.arange(CS, dtype=jnp.int32)
    dump_col = n + jnp.arange(CS, dtype=jnp.int32)

    def _core_layout(chunks, r_real, fill):
        part = chunks.reshape(NTILES, r_real, CS)
        if r_real == nmx:
            return part
        extra = jnp.broadcast_to(fill, (NTILES, nmx - r_real, CS))
        return jnp.concatenate([part, extra], axis=1)

    row3a = jnp.concatenate([
        _core_layout(chr_[:NTILES * r0], r0, dump_row),
        _core_layout(chr_[NTILES * r0:], r1, dump_row)], axis=0)
    col3a = jnp.concatenate([
        _core_layout(chc_[:NTILES * r0], r0, dump_col),
        _core_layout(chc_[NTILES * r0:], r1, dump_col)], axis=0)

    degc = _sc_degree(col3, ones_h, zagg, n_pad, nch, rpt, d)
    degs = degc[:, :n, :16]
    bl = 2000
    xs = _tc_scale(x, degs, bl)
    P = _sc_aggregate(xs, row3a, col3a, zagg, n_pad, nmx, rpt, d)[:, :n, :]

    Wdp = jnp.zeros((d, d), jnp.float32).at[:, :na].set(Wd)
    bdp = jnp.zeros((1, d), jnp.float32).at[0, :na].set(bd)
    return _tc_dense(x, xs, P, degs, Ws, bs, Wdp, bdp, bl)
